# Initial kernel scaffold; baseline (speedup 1.0000x reference)
import jax
import jax.numpy as jnp
from jax.experimental import pallas as pl


def kernel(tens_indices, tens_values):
    raise NotImplementedError("write your pallas kernel here")


# speedup vs baseline: 5.0670x; 5.0670x over previous
import functools

import jax
import jax.numpy as jnp
from jax import lax
from jax.experimental import pallas as pl
from jax.experimental.pallas import tpu as pltpu
from jax.experimental.pallas import tpu_sc as plsc

N_SEG = 16384
NNZ = 268435
D = 64
EPS = 1e-16

NC = 2
NS = 16
LANES = 16
DH = D // NC

BLK = 128
SB = 1024
NSB = -(-NNZ // SB)
TAIL = NNZ - (NSB - 1) * SB
NPAD = NSB * SB
ITERS = -(-NSB // NS)
TROWS = N_SEG + BLK
NCHUNK = TROWS // BLK
SEG_PT = N_SEG // NS


def _body(seg_hbm, vals_hbm, out_hbm,
          table_sh, counts_sh, idx_all, vbuf, cbuf, tbuf, zbuf, zflat, ones_b):
    cid = lax.axis_index("c")
    sid = lax.axis_index("s")
    dcol = cid * DH

    z16 = jnp.zeros((LANES,), jnp.float32)
    one16 = jnp.ones((LANES,), jnp.float32)
    for k in range(BLK // LANES):
        zflat[pl.ds(k * LANES, LANES)] = z16
        ones_b[pl.ds(k * LANES, LANES)] = one16

    def _zrow(r, c):
        zbuf[r, pl.ds(0, LANES)] = z16
        zbuf[r, pl.ds(LANES, LANES)] = z16
        return c
    lax.fori_loop(0, BLK, _zrow, 0)

    for i in range(-(-NCHUNK // NS)):
        c = sid + NS * i

        @pl.when(c < NCHUNK)
        def _():
            pltpu.sync_copy(zbuf, table_sh.at[pl.ds(c * BLK, BLK)])
            pltpu.sync_copy(zflat, counts_sh.at[pl.ds(c * BLK, BLK)])

    plsc.subcore_barrier()

    def _p1(i, carry):
        s = sid + NS * i

        @pl.when(s < NSB)
        def _():
            pltpu.sync_copy(seg_hbm.at[pl.ds(s * (SB // BLK), SB // BLK)],
                            idx_all.at[pl.ds(i * (SB // BLK), SB // BLK)])

            @pl.when(s < NSB - 1)
            def _():
                pltpu.sync_copy(
                    vals_hbm.at[pl.ds(s * SB, SB), pl.ds(dcol, DH)], vbuf)

            @pl.when(s == NSB - 1)
            def _():
                pltpu.sync_copy(
                    vals_hbm.at[pl.ds((NSB - 1) * SB, TAIL), pl.ds(dcol, DH)],
                    vbuf.at[pl.ds(0, TAIL)])

            for j in range(SB // BLK):
                idx_row = idx_all.at[i * (SB // BLK) + j]
                pltpu.sync_copy(vbuf.at[pl.ds(j * BLK, BLK)],
                                table_sh.at[idx_row], add=True)
                pltpu.sync_copy(ones_b, counts_sh.at[idx_row], add=True)
        return carry
    lax.fori_loop(0, ITERS, _p1, 0)

    plsc.subcore_barrier()

    r0 = sid * SEG_PT
    pltpu.sync_copy(counts_sh.at[pl.ds(r0, SEG_PT)], cbuf)

    def _inv(k, carry):
        v = cbuf[pl.ds(k * LANES, LANES)]
        cbuf[pl.ds(k * LANES, LANES)] = 1.0 / (v + EPS)
        return carry
    lax.fori_loop(0, SEG_PT // LANES, _inv, 0)

    def _chunk(c, carry):
        pltpu.sync_copy(table_sh.at[pl.ds(r0 + c * BLK, BLK)], tbuf)

        def _grp(g, carry2):
            ivec = cbuf[pl.ds(c * BLK + g * LANES, LANES)]
            for r16 in range(LANES):
                bv = jnp.full((LANES,), ivec[r16], jnp.float32)
                r = g * LANES + r16
                tbuf[r, pl.ds(0, LANES)] = tbuf[r, pl.ds(0, LANES)] * bv
                tbuf[r, pl.ds(LANES, LANES)] = (
                    tbuf[r, pl.ds(LANES, LANES)] * bv)
            return carry2
        lax.fori_loop(0, BLK // LANES, _grp, 0)
        pltpu.sync_copy(tbuf, table_sh.at[pl.ds(r0 + c * BLK, BLK)])
        return carry
    lax.fori_loop(0, SEG_PT // BLK, _chunk, 0)

    plsc.subcore_barrier()

    def _p2(i, carry):
        s = sid + NS * i

        @pl.when(s < NSB)
        def _():
            for j in range(SB // BLK):
                idx_row = idx_all.at[i * (SB // BLK) + j]
                pltpu.sync_copy(table_sh.at[idx_row],
                                vbuf.at[pl.ds(j * BLK, BLK)])

            @pl.when(s < NSB - 1)
            def _():
                pltpu.sync_copy(
                    vbuf, out_hbm.at[pl.ds(s * SB, SB), pl.ds(dcol, DH)])

            @pl.when(s == NSB - 1)
            def _():
                pltpu.sync_copy(
                    vbuf.at[pl.ds(0, TAIL)],
                    out_hbm.at[pl.ds((NSB - 1) * SB, TAIL), pl.ds(dcol, DH)])
        return carry
    lax.fori_loop(0, ITERS, _p2, 0)


@jax.jit
def _pool(seg2d, vals):
    mesh = plsc.VectorSubcoreMesh(core_axis_name="c", subcore_axis_name="s")
    run = pl.kernel(
        _body,
        out_type=jax.ShapeDtypeStruct((NNZ, D), jnp.float32),
        mesh=mesh,
        compiler_params=pltpu.CompilerParams(use_tc_tiling_on_sc=False),
        scratch_types=[
            pltpu.VMEM_SHARED((TROWS, DH), jnp.float32),
            pltpu.VMEM_SHARED((TROWS,), jnp.float32),
            pltpu.VMEM((ITERS * (SB // BLK), BLK), jnp.int32),
            pltpu.VMEM((SB, DH), jnp.float32),
            pltpu.VMEM((SEG_PT,), jnp.float32),
            pltpu.VMEM((BLK, DH), jnp.float32),
            pltpu.VMEM((BLK, DH), jnp.float32),
            pltpu.VMEM((BLK,), jnp.float32),
            pltpu.VMEM((BLK,), jnp.float32),
        ],
    )
    return run(seg2d, vals)


def kernel(tens_indices, tens_values):
    seg = tens_indices[1].astype(jnp.int32)
    seg_pad = jnp.full((NPAD,), N_SEG, jnp.int32).at[:NNZ].set(seg)
    seg2d = seg_pad.reshape(NPAD // BLK, BLK)
    return _pool(seg2d, tens_values)


# speedup vs baseline: 5.0854x; 1.0036x over previous
import functools

import jax
import jax.numpy as jnp
from jax import lax
from jax.experimental import pallas as pl
from jax.experimental.pallas import tpu as pltpu
from jax.experimental.pallas import tpu_sc as plsc

N_SEG = 16384
NNZ = 268435
D = 64
EPS = 1e-16

NC = 2
NS = 16
LANES = 16
DH = D // NC

BLK = 128
SB = 2048
JB = SB // BLK
NSB = -(-NNZ // SB)
TAIL = NNZ - (NSB - 1) * SB
ITERS = -(-NSB // NS)
NPAD = ITERS * NS * SB
TROWS = N_SEG + BLK
NCHUNK = TROWS // BLK
SEG_PT = N_SEG // NS


def _body(seg_hbm, vals_hbm, out_hbm,
          table_sh, counts_sh, idx_all, vbuf, cbuf, tbuf, zbuf, zflat, ones_b):
    cid = lax.axis_index("c")
    sid = lax.axis_index("s")
    dcol = cid * DH

    z16 = jnp.zeros((LANES,), jnp.float32)
    one16 = jnp.ones((LANES,), jnp.float32)
    for k in range(BLK // LANES):
        zflat[pl.ds(k * LANES, LANES)] = z16
        ones_b[pl.ds(k * LANES, LANES)] = one16

    def _zrow(r, c):
        zbuf[r, pl.ds(0, LANES)] = z16
        zbuf[r, pl.ds(LANES, LANES)] = z16
        return c
    lax.fori_loop(0, BLK, _zrow, 0)

    for i in range(-(-NCHUNK // NS)):
        c = sid + NS * i

        @pl.when(c < NCHUNK)
        def _():
            pltpu.sync_copy(zbuf, table_sh.at[pl.ds(c * BLK, BLK)])
            pltpu.sync_copy(zflat, counts_sh.at[pl.ds(c * BLK, BLK)])

    plsc.subcore_barrier()

    def _p1(i, carry):
        s = sid + NS * i

        @pl.when(s < NSB)
        def _():
            pltpu.sync_copy(seg_hbm.at[pl.ds(s * JB, JB)],
                            idx_all.at[pl.ds(i * JB, JB)])

            @pl.when(s < NSB - 1)
            def _():
                pltpu.sync_copy(
                    vals_hbm.at[pl.ds(s * SB, SB), pl.ds(dcol, DH)], vbuf)

            @pl.when(s == NSB - 1)
            def _():
                pltpu.sync_copy(
                    vals_hbm.at[pl.ds((NSB - 1) * SB, TAIL), pl.ds(dcol, DH)],
                    vbuf.at[pl.ds(0, TAIL)])

            for j in range(JB):
                idx_row = idx_all.at[i * JB + j]
                pltpu.sync_copy(vbuf.at[pl.ds(j * BLK, BLK)],
                                table_sh.at[idx_row], add=True)
                pltpu.sync_copy(ones_b, counts_sh.at[idx_row], add=True)
        return carry
    lax.fori_loop(0, ITERS, _p1, 0)

    plsc.subcore_barrier()

    r0 = sid * SEG_PT
    pltpu.sync_copy(counts_sh.at[pl.ds(r0, SEG_PT)], cbuf)

    def _inv(k, carry):
        v = cbuf[pl.ds(k * LANES, LANES)]
        cbuf[pl.ds(k * LANES, LANES)] = 1.0 / (v + EPS)
        return carry
    lax.fori_loop(0, SEG_PT // LANES, _inv, 0)

    def _chunk(c, carry):
        pltpu.sync_copy(table_sh.at[pl.ds(r0 + c * BLK, BLK)], tbuf)

        def _grp(g, carry2):
            ivec = cbuf[pl.ds(c * BLK + g * LANES, LANES)]
            for r16 in range(LANES):
                bv = jnp.full((LANES,), ivec[r16], jnp.float32)
                r = g * LANES + r16
                tbuf[r, pl.ds(0, LANES)] = tbuf[r, pl.ds(0, LANES)] * bv
                tbuf[r, pl.ds(LANES, LANES)] = (
                    tbuf[r, pl.ds(LANES, LANES)] * bv)
            return carry2
        lax.fori_loop(0, BLK // LANES, _grp, 0)
        pltpu.sync_copy(tbuf, table_sh.at[pl.ds(r0 + c * BLK, BLK)])
        return carry
    lax.fori_loop(0, SEG_PT // BLK, _chunk, 0)

    plsc.subcore_barrier()

    def _p2(i, carry):
        s = sid + NS * i

        @pl.when(s < NSB)
        def _():
            for j in range(JB):
                idx_row = idx_all.at[i * JB + j]
                pltpu.sync_copy(table_sh.at[idx_row],
                                vbuf.at[pl.ds(j * BLK, BLK)])

            @pl.when(s < NSB - 1)
            def _():
                pltpu.sync_copy(
                    vbuf, out_hbm.at[pl.ds(s * SB, SB), pl.ds(dcol, DH)])

            @pl.when(s == NSB - 1)
            def _():
                pltpu.sync_copy(
                    vbuf.at[pl.ds(0, TAIL)],
                    out_hbm.at[pl.ds((NSB - 1) * SB, TAIL), pl.ds(dcol, DH)])
        return carry
    lax.fori_loop(0, ITERS, _p2, 0)


@jax.jit
def _pool(seg2d, vals):
    mesh = plsc.VectorSubcoreMesh(core_axis_name="c", subcore_axis_name="s")
    run = pl.kernel(
        _body,
        out_type=jax.ShapeDtypeStruct((NNZ, D), jnp.float32),
        mesh=mesh,
        compiler_params=pltpu.CompilerParams(use_tc_tiling_on_sc=False),
        scratch_types=[
            pltpu.VMEM_SHARED((TROWS, DH), jnp.float32),
            pltpu.VMEM_SHARED((TROWS,), jnp.float32),
            pltpu.VMEM((ITERS * JB, BLK), jnp.int32),
            pltpu.VMEM((SB, DH), jnp.float32),
            pltpu.VMEM((SEG_PT,), jnp.float32),
            pltpu.VMEM((BLK, DH), jnp.float32),
            pltpu.VMEM((BLK, DH), jnp.float32),
            pltpu.VMEM((BLK,), jnp.float32),
            pltpu.VMEM((BLK,), jnp.float32),
        ],
    )
    return run(seg2d, vals)


def kernel(tens_indices, tens_values):
    seg = tens_indices[1].astype(jnp.int32)
    seg_pad = jnp.full((NPAD,), N_SEG, jnp.int32).at[:NNZ].set(seg)
    seg2d = seg_pad.reshape(NPAD // BLK, BLK)
    return _pool(seg2d, tens_values)


# speedup vs baseline: 5.3245x; 1.0470x over previous
import functools

import jax
import jax.numpy as jnp
from jax import lax
from jax.experimental import pallas as pl
from jax.experimental.pallas import tpu as pltpu
from jax.experimental.pallas import tpu_sc as plsc

N_SEG = 16384
NNZ = 268435
D = 64
EPS = 1e-16

NC = 2
NS = 16
LANES = 16
DH = D // NC

BLK = 2048
SB = 2048
RC = 128
JB = SB // BLK
NSB = -(-NNZ // SB)
TAIL = NNZ - (NSB - 1) * SB
ITERS = -(-NSB // NS)
NPAD = ITERS * NS * SB
TROWS = N_SEG
NCHUNK = TROWS // RC
SEG_PT = N_SEG // NS


def _body(seg_hbm, vals_hbm, out_hbm,
          table_sh, counts_sh, idx_all, vbuf, cbuf, tbuf, zflat, ones_b,
          ones_t):
    cid = lax.axis_index("c")
    sid = lax.axis_index("s")
    dcol = cid * DH

    z16 = jnp.zeros((LANES,), jnp.float32)
    one16 = jnp.ones((LANES,), jnp.float32)
    lane = jnp.arange(LANES, dtype=jnp.int32)

    def _ones(k, c):
        ones_b[pl.ds(k * LANES, LANES)] = one16
        tv = jnp.where(lane + k * LANES < TAIL, 1.0, 0.0).astype(jnp.float32)
        ones_t[pl.ds(k * LANES, LANES)] = tv
        return c
    lax.fori_loop(0, BLK // LANES, _ones, 0)

    def _zrow(r, c):
        tbuf[r, pl.ds(0, LANES)] = z16
        tbuf[r, pl.ds(LANES, LANES)] = z16
        return c
    lax.fori_loop(0, RC, _zrow, 0)
    for k in range(RC // LANES):
        zflat[pl.ds(k * LANES, LANES)] = z16

    for i in range(-(-NCHUNK // NS)):
        c = sid + NS * i

        @pl.when(c < NCHUNK)
        def _():
            pltpu.sync_copy(tbuf, table_sh.at[pl.ds(c * RC, RC)])
            pltpu.sync_copy(zflat, counts_sh.at[pl.ds(c * RC, RC)])

    plsc.subcore_barrier()

    def _p1(i, carry):
        s = sid + NS * i

        @pl.when(s < NSB)
        def _():
            pltpu.sync_copy(seg_hbm.at[pl.ds(s * JB, JB)],
                            idx_all.at[pl.ds(i * JB, JB)])

            @pl.when(s < NSB - 1)
            def _():
                pltpu.sync_copy(
                    vals_hbm.at[pl.ds(s * SB, SB), pl.ds(dcol, DH)], vbuf)

            @pl.when(s == NSB - 1)
            def _():
                pltpu.sync_copy(
                    vals_hbm.at[pl.ds((NSB - 1) * SB, TAIL), pl.ds(dcol, DH)],
                    vbuf.at[pl.ds(0, TAIL)])

                def _ztail(r, c2):
                    vbuf[TAIL + r, pl.ds(0, LANES)] = z16
                    vbuf[TAIL + r, pl.ds(LANES, LANES)] = z16
                    return c2
                lax.fori_loop(0, SB - TAIL, _ztail, 0)

            idx_row = idx_all.at[i]
            pltpu.sync_copy(vbuf, table_sh.at[idx_row], add=True)

            @pl.when(s < NSB - 1)
            def _():
                pltpu.sync_copy(ones_b, counts_sh.at[idx_row], add=True)

            @pl.when(s == NSB - 1)
            def _():
                pltpu.sync_copy(ones_t, counts_sh.at[idx_row], add=True)
        return carry
    lax.fori_loop(0, ITERS, _p1, 0)

    plsc.subcore_barrier()

    r0 = sid * SEG_PT
    pltpu.sync_copy(counts_sh.at[pl.ds(r0, SEG_PT)], cbuf)

    def _inv(k, carry):
        v = cbuf[pl.ds(k * LANES, LANES)]
        cbuf[pl.ds(k * LANES, LANES)] = 1.0 / (v + EPS)
        return carry
    lax.fori_loop(0, SEG_PT // LANES, _inv, 0)

    def _chunk(c, carry):
        pltpu.sync_copy(table_sh.at[pl.ds(r0 + c * RC, RC)], tbuf)

        def _grp(g, carry2):
            ivec = cbuf[pl.ds(c * RC + g * LANES, LANES)]
            for r16 in range(LANES):
                bv = jnp.full((LANES,), ivec[r16], jnp.float32)
                r = g * LANES + r16
                tbuf[r, pl.ds(0, LANES)] = tbuf[r, pl.ds(0, LANES)] * bv
                tbuf[r, pl.ds(LANES, LANES)] = (
                    tbuf[r, pl.ds(LANES, LANES)] * bv)
            return carry2
        lax.fori_loop(0, RC // LANES, _grp, 0)
        pltpu.sync_copy(tbuf, table_sh.at[pl.ds(r0 + c * RC, RC)])
        return carry
    lax.fori_loop(0, SEG_PT // RC, _chunk, 0)

    plsc.subcore_barrier()

    def _p2(i, carry):
        s = sid + NS * i

        @pl.when(s < NSB)
        def _():
            idx_row = idx_all.at[i]
            pltpu.sync_copy(table_sh.at[idx_row], vbuf)

            @pl.when(s < NSB - 1)
            def _():
                pltpu.sync_copy(
                    vbuf, out_hbm.at[pl.ds(s * SB, SB), pl.ds(dcol, DH)])

            @pl.when(s == NSB - 1)
            def _():
                pltpu.sync_copy(
                    vbuf.at[pl.ds(0, TAIL)],
                    out_hbm.at[pl.ds((NSB - 1) * SB, TAIL), pl.ds(dcol, DH)])
        return carry
    lax.fori_loop(0, ITERS, _p2, 0)


@jax.jit
def _pool(seg2d, vals):
    mesh = plsc.VectorSubcoreMesh(core_axis_name="c", subcore_axis_name="s")
    run = pl.kernel(
        _body,
        out_type=jax.ShapeDtypeStruct((NNZ, D), jnp.float32),
        mesh=mesh,
        compiler_params=pltpu.CompilerParams(use_tc_tiling_on_sc=False),
        scratch_types=[
            pltpu.VMEM_SHARED((TROWS, DH), jnp.float32),
            pltpu.VMEM_SHARED((TROWS,), jnp.float32),
            pltpu.VMEM((ITERS * JB, BLK), jnp.int32),
            pltpu.VMEM((SB, DH), jnp.float32),
            pltpu.VMEM((SEG_PT,), jnp.float32),
            pltpu.VMEM((RC, DH), jnp.float32),
            pltpu.VMEM((RC,), jnp.float32),
            pltpu.VMEM((BLK,), jnp.float32),
            pltpu.VMEM((BLK,), jnp.float32),
        ],
    )
    return run(seg2d, vals)


def kernel(tens_indices, tens_values):
    seg = tens_indices[1].astype(jnp.int32)
    seg_pad = jnp.zeros((NPAD,), jnp.int32).at[:NNZ].set(seg)
    seg2d = seg_pad.reshape(NPAD // BLK, BLK)
    return _pool(seg2d, tens_values)


# speedup vs baseline: 5.6459x; 1.0604x over previous
import functools

import jax
import jax.numpy as jnp
from jax import lax
from jax.experimental import pallas as pl
from jax.experimental.pallas import tpu as pltpu
from jax.experimental.pallas import tpu_sc as plsc

N_SEG = 16384
NNZ = 268435
D = 64
EPS = 1e-16

NC = 2
NS = 16
LANES = 16
DH = D // NC

BLK = 1024
SB = 1024
RC = 128
JB = SB // BLK
NSB = -(-NNZ // SB)
TAIL = NNZ - (NSB - 1) * SB
ITERS = -(-NSB // NS)
NPAD = ITERS * NS * SB
TROWS = N_SEG
NCHUNK = TROWS // RC
SEG_PT = N_SEG // NS


def _body(seg_hbm, vals_hbm, out_hbm,
          table_sh, counts_sh, idx_all, vbuf, cbuf, tbuf, zflat, ones_b,
          ones_t, vsem0, vsem1):
    cid = lax.axis_index("c")
    sid = lax.axis_index("s")
    dcol = cid * DH

    z16 = jnp.zeros((LANES,), jnp.float32)
    one16 = jnp.ones((LANES,), jnp.float32)
    lane = jnp.arange(LANES, dtype=jnp.int32)

    def _ones(k, c):
        ones_b[pl.ds(k * LANES, LANES)] = one16
        tv = jnp.where(lane + k * LANES < TAIL, 1.0, 0.0).astype(jnp.float32)
        ones_t[pl.ds(k * LANES, LANES)] = tv
        return c
    lax.fori_loop(0, BLK // LANES, _ones, 0)

    def _zrow(r, c):
        tbuf[r, pl.ds(0, LANES)] = z16
        tbuf[r, pl.ds(LANES, LANES)] = z16
        return c
    lax.fori_loop(0, RC, _zrow, 0)
    for k in range(RC // LANES):
        zflat[pl.ds(k * LANES, LANES)] = z16

    for i in range(-(-NCHUNK // NS)):
        c = sid + NS * i

        @pl.when(c < NCHUNK)
        def _():
            pltpu.sync_copy(tbuf, table_sh.at[pl.ds(c * RC, RC)])
            pltpu.sync_copy(zflat, counts_sh.at[pl.ds(c * RC, RC)])

    plsc.subcore_barrier()

    vsem = (vsem0, vsem1)

    def _vals_args(i, b, tail):
        s = sid + NS * i
        if tail:
            src = vals_hbm.at[pl.ds((NSB - 1) * SB, TAIL), pl.ds(dcol, DH)]
            dst = vbuf.at[b, pl.ds(0, TAIL)]
        else:
            src = vals_hbm.at[pl.ds(s * SB, SB), pl.ds(dcol, DH)]
            dst = vbuf.at[b]
        return (src, dst, vsem[b])

    def _fire_vals(i, b):
        s = sid + NS * i

        @pl.when(s < NSB - 1)
        def _():
            pltpu.async_copy(*_vals_args(i, b, False))

        @pl.when(s == NSB - 1)
        def _():
            pltpu.async_copy(*_vals_args(i, b, True))

    def _wait_vals(i, b):
        s = sid + NS * i

        @pl.when(s < NSB - 1)
        def _():
            pltpu.make_async_copy(*_vals_args(i, b, False)).wait()

        @pl.when(s == NSB - 1)
        def _():
            pltpu.make_async_copy(*_vals_args(i, b, True)).wait()

    def _load_idx(i):
        s = sid + NS * i

        @pl.when(s < NSB)
        def _():
            pltpu.sync_copy(seg_hbm.at[pl.ds(s * JB, JB)],
                            idx_all.at[pl.ds(i * JB, JB)])

    def _scatter(i, b):
        s = sid + NS * i

        @pl.when(s < NSB)
        def _():
            @pl.when(s == NSB - 1)
            def _():
                def _ztail(r, c2):
                    vbuf[b, TAIL + r, pl.ds(0, LANES)] = z16
                    vbuf[b, TAIL + r, pl.ds(LANES, LANES)] = z16
                    return c2
                lax.fori_loop(0, SB - TAIL, _ztail, 0)

            idx_row = idx_all.at[i]
            pltpu.sync_copy(vbuf.at[b], table_sh.at[idx_row], add=True)

            @pl.when(s < NSB - 1)
            def _():
                pltpu.sync_copy(ones_b, counts_sh.at[idx_row], add=True)

            @pl.when(s == NSB - 1)
            def _():
                pltpu.sync_copy(ones_t, counts_sh.at[idx_row], add=True)

    _fire_vals(0, 0)

    def _p1(k, carry):
        i0 = 2 * k
        _load_idx(i0)
        _wait_vals(i0, 0)
        _fire_vals(i0 + 1, 1)
        _scatter(i0, 0)
        _load_idx(i0 + 1)
        _wait_vals(i0 + 1, 1)
        _fire_vals(i0 + 2, 0)
        _scatter(i0 + 1, 1)
        return carry
    lax.fori_loop(0, (ITERS + 1) // 2, _p1, 0)

    plsc.subcore_barrier()

    r0 = sid * SEG_PT
    pltpu.sync_copy(counts_sh.at[pl.ds(r0, SEG_PT)], cbuf)

    def _inv(k, carry):
        v = cbuf[pl.ds(k * LANES, LANES)]
        cbuf[pl.ds(k * LANES, LANES)] = 1.0 / (v + EPS)
        return carry
    lax.fori_loop(0, SEG_PT // LANES, _inv, 0)

    def _chunk(c, carry):
        pltpu.sync_copy(table_sh.at[pl.ds(r0 + c * RC, RC)], tbuf)

        def _grp(g, carry2):
            ivec = cbuf[pl.ds(c * RC + g * LANES, LANES)]
            for r16 in range(LANES):
                bv = jnp.full((LANES,), ivec[r16], jnp.float32)
                r = g * LANES + r16
                tbuf[r, pl.ds(0, LANES)] = tbuf[r, pl.ds(0, LANES)] * bv
                tbuf[r, pl.ds(LANES, LANES)] = (
                    tbuf[r, pl.ds(LANES, LANES)] * bv)
            return carry2
        lax.fori_loop(0, RC // LANES, _grp, 0)
        pltpu.sync_copy(tbuf, table_sh.at[pl.ds(r0 + c * RC, RC)])
        return carry
    lax.fori_loop(0, SEG_PT // RC, _chunk, 0)

    plsc.subcore_barrier()

    def _p2(i, carry):
        s = sid + NS * i

        @pl.when(s < NSB)
        def _():
            idx_row = idx_all.at[i]
            pltpu.sync_copy(table_sh.at[idx_row], vbuf.at[0])

            @pl.when(s < NSB - 1)
            def _():
                pltpu.sync_copy(
                    vbuf.at[0], out_hbm.at[pl.ds(s * SB, SB), pl.ds(dcol, DH)])

            @pl.when(s == NSB - 1)
            def _():
                pltpu.sync_copy(
                    vbuf.at[0, pl.ds(0, TAIL)],
                    out_hbm.at[pl.ds((NSB - 1) * SB, TAIL), pl.ds(dcol, DH)])
        return carry
    lax.fori_loop(0, ITERS, _p2, 0)


@jax.jit
def _pool(seg2d, vals):
    mesh = plsc.VectorSubcoreMesh(core_axis_name="c", subcore_axis_name="s")
    run = pl.kernel(
        _body,
        out_type=jax.ShapeDtypeStruct((NNZ, D), jnp.float32),
        mesh=mesh,
        compiler_params=pltpu.CompilerParams(use_tc_tiling_on_sc=False),
        scratch_types=[
            pltpu.VMEM_SHARED((TROWS, DH), jnp.float32),
            pltpu.VMEM_SHARED((TROWS,), jnp.float32),
            pltpu.VMEM((ITERS * JB, BLK), jnp.int32),
            pltpu.VMEM((2, SB, DH), jnp.float32),
            pltpu.VMEM((SEG_PT,), jnp.float32),
            pltpu.VMEM((RC, DH), jnp.float32),
            pltpu.VMEM((RC,), jnp.float32),
            pltpu.VMEM((BLK,), jnp.float32),
            pltpu.VMEM((BLK,), jnp.float32),
            pltpu.SemaphoreType.DMA,
            pltpu.SemaphoreType.DMA,
        ],
    )
    return run(seg2d, vals)


def kernel(tens_indices, tens_values):
    seg = tens_indices[1].astype(jnp.int32)
    seg_pad = jnp.zeros((NPAD,), jnp.int32).at[:NNZ].set(seg)
    seg2d = seg_pad.reshape(NPAD // BLK, BLK)
    return _pool(seg2d, tens_values)


# speedup vs baseline: 5.8592x; 1.0378x over previous
import functools

import jax
import jax.numpy as jnp
from jax import lax
from jax.experimental import pallas as pl
from jax.experimental.pallas import tpu as pltpu
from jax.experimental.pallas import tpu_sc as plsc

N_SEG = 16384
NNZ = 268435
D = 64
EPS = 1e-16

NC = 2
NS = 16
LANES = 16
DH = D // NC

BLK = 1024
SB = 1024
RC = 128
JB = SB // BLK
NSB = -(-NNZ // SB)
TAIL = NNZ - (NSB - 1) * SB
ITERS = -(-NSB // NS)
NPAD = ITERS * NS * SB
TROWS = N_SEG
NCHUNK = TROWS // RC
SEG_PT = N_SEG // NS


def _body(seg_hbm, vals_hbm, out_hbm,
          table_sh, counts_sh, idx_all, vbuf, cbuf, tbuf, zflat, ones_b,
          ones_t, vsem0, vsem1):
    cid = lax.axis_index("c")
    sid = lax.axis_index("s")
    dcol = cid * DH

    z16 = jnp.zeros((LANES,), jnp.float32)
    one16 = jnp.ones((LANES,), jnp.float32)
    lane = jnp.arange(LANES, dtype=jnp.int32)

    def _ones(k, c):
        ones_b[pl.ds(k * LANES, LANES)] = one16
        tv = jnp.where(lane + k * LANES < TAIL, 1.0, 0.0).astype(jnp.float32)
        ones_t[pl.ds(k * LANES, LANES)] = tv
        return c
    lax.fori_loop(0, BLK // LANES, _ones, 0)

    def _zrow(r, c):
        tbuf[r, pl.ds(0, LANES)] = z16
        tbuf[r, pl.ds(LANES, LANES)] = z16
        return c
    lax.fori_loop(0, RC, _zrow, 0)
    for k in range(RC // LANES):
        zflat[pl.ds(k * LANES, LANES)] = z16

    for i in range(-(-NCHUNK // NS)):
        c = sid + NS * i

        @pl.when(c < NCHUNK)
        def _():
            pltpu.sync_copy(tbuf, table_sh.at[pl.ds(c * RC, RC)])
            pltpu.sync_copy(zflat, counts_sh.at[pl.ds(c * RC, RC)])

    plsc.subcore_barrier()

    vsem = (vsem0, vsem1)

    def _vals_args(i, b, tail):
        s = sid + NS * i
        if tail:
            src = vals_hbm.at[pl.ds((NSB - 1) * SB, TAIL), pl.ds(dcol, DH)]
            dst = vbuf.at[b, pl.ds(0, TAIL)]
        else:
            src = vals_hbm.at[pl.ds(s * SB, SB), pl.ds(dcol, DH)]
            dst = vbuf.at[b]
        return (src, dst, vsem[b])

    def _fire_vals(i, b):
        s = sid + NS * i

        @pl.when(s < NSB - 1)
        def _():
            pltpu.async_copy(*_vals_args(i, b, False))

        @pl.when(s == NSB - 1)
        def _():
            pltpu.async_copy(*_vals_args(i, b, True))

    def _wait_vals(i, b):
        s = sid + NS * i

        @pl.when(s < NSB - 1)
        def _():
            pltpu.make_async_copy(*_vals_args(i, b, False)).wait()

        @pl.when(s == NSB - 1)
        def _():
            pltpu.make_async_copy(*_vals_args(i, b, True)).wait()

    def _load_idx(i):
        s = sid + NS * i

        @pl.when(s < NSB)
        def _():
            pltpu.sync_copy(seg_hbm.at[pl.ds(s * JB, JB)],
                            idx_all.at[pl.ds(i * JB, JB)])

    def _scatter(i, b):
        s = sid + NS * i

        @pl.when(s < NSB)
        def _():
            @pl.when(s == NSB - 1)
            def _():
                def _ztail(r, c2):
                    vbuf[b, TAIL + r, pl.ds(0, LANES)] = z16
                    vbuf[b, TAIL + r, pl.ds(LANES, LANES)] = z16
                    return c2
                lax.fori_loop(0, SB - TAIL, _ztail, 0)

            idx_row = idx_all.at[i]
            pltpu.sync_copy(vbuf.at[b], table_sh.at[idx_row], add=True)

            @pl.when(s < NSB - 1)
            def _():
                pltpu.sync_copy(ones_b, counts_sh.at[idx_row], add=True)

            @pl.when(s == NSB - 1)
            def _():
                pltpu.sync_copy(ones_t, counts_sh.at[idx_row], add=True)

    _fire_vals(0, 0)

    def _p1(k, carry):
        i0 = 2 * k
        _load_idx(i0)
        _wait_vals(i0, 0)
        _fire_vals(i0 + 1, 1)
        _scatter(i0, 0)
        _load_idx(i0 + 1)
        _wait_vals(i0 + 1, 1)
        _fire_vals(i0 + 2, 0)
        _scatter(i0 + 1, 1)
        return carry
    lax.fori_loop(0, (ITERS + 1) // 2, _p1, 0)

    plsc.subcore_barrier()

    r0 = sid * SEG_PT
    pltpu.sync_copy(counts_sh.at[pl.ds(r0, SEG_PT)], cbuf)

    def _inv(k, carry):
        v = cbuf[pl.ds(k * LANES, LANES)]
        cbuf[pl.ds(k * LANES, LANES)] = 1.0 / (v + EPS)
        return carry
    lax.fori_loop(0, SEG_PT // LANES, _inv, 0)

    def _chunk(c, carry):
        pltpu.sync_copy(table_sh.at[pl.ds(r0 + c * RC, RC)], tbuf)

        def _grp(g, carry2):
            ivec = cbuf[pl.ds(c * RC + g * LANES, LANES)]
            for r16 in range(LANES):
                bv = jnp.full((LANES,), ivec[r16], jnp.float32)
                r = g * LANES + r16
                tbuf[r, pl.ds(0, LANES)] = tbuf[r, pl.ds(0, LANES)] * bv
                tbuf[r, pl.ds(LANES, LANES)] = (
                    tbuf[r, pl.ds(LANES, LANES)] * bv)
            return carry2
        lax.fori_loop(0, RC // LANES, _grp, 0)
        pltpu.sync_copy(tbuf, table_sh.at[pl.ds(r0 + c * RC, RC)])
        return carry
    lax.fori_loop(0, SEG_PT // RC, _chunk, 0)

    plsc.subcore_barrier()

    def _out_args(i, b, tail):
        s = sid + NS * i
        if tail:
            src = vbuf.at[b, pl.ds(0, TAIL)]
            dst = out_hbm.at[pl.ds((NSB - 1) * SB, TAIL), pl.ds(dcol, DH)]
        else:
            src = vbuf.at[b]
            dst = out_hbm.at[pl.ds(s * SB, SB), pl.ds(dcol, DH)]
        return (src, dst, vsem[b])

    def _wait_out(i, b):
        s = sid + NS * i

        @pl.when(s < NSB - 1)
        def _():
            pltpu.make_async_copy(*_out_args(i, b, False)).wait()

        @pl.when(s == NSB - 1)
        def _():
            pltpu.make_async_copy(*_out_args(i, b, True)).wait()

    def _gather_store(i, b):
        s = sid + NS * i

        @pl.when(s < NSB)
        def _():
            idx_row = idx_all.at[i]
            pltpu.sync_copy(table_sh.at[idx_row], vbuf.at[b])

            @pl.when(s < NSB - 1)
            def _():
                pltpu.async_copy(*_out_args(i, b, False))

            @pl.when(s == NSB - 1)
            def _():
                pltpu.async_copy(*_out_args(i, b, True))

    def _p2(k, carry):
        i0 = 2 * k
        _gather_store(i0, 0)

        @pl.when(i0 >= 1)
        def _():
            _wait_out(i0 - 1, 1)
        _gather_store(i0 + 1, 1)
        _wait_out(i0, 0)
        return carry
    lax.fori_loop(0, (ITERS + 1) // 2, _p2, 0)


@jax.jit
def _pool(seg2d, vals):
    mesh = plsc.VectorSubcoreMesh(core_axis_name="c", subcore_axis_name="s")
    run = pl.kernel(
        _body,
        out_type=jax.ShapeDtypeStruct((NNZ, D), jnp.float32),
        mesh=mesh,
        compiler_params=pltpu.CompilerParams(use_tc_tiling_on_sc=False),
        scratch_types=[
            pltpu.VMEM_SHARED((TROWS, DH), jnp.float32),
            pltpu.VMEM_SHARED((TROWS,), jnp.float32),
            pltpu.VMEM((ITERS * JB, BLK), jnp.int32),
            pltpu.VMEM((2, SB, DH), jnp.float32),
            pltpu.VMEM((SEG_PT,), jnp.float32),
            pltpu.VMEM((RC, DH), jnp.float32),
            pltpu.VMEM((RC,), jnp.float32),
            pltpu.VMEM((BLK,), jnp.float32),
            pltpu.VMEM((BLK,), jnp.float32),
            pltpu.SemaphoreType.DMA,
            pltpu.SemaphoreType.DMA,
        ],
    )
    return run(seg2d, vals)


def kernel(tens_indices, tens_values):
    seg = tens_indices[1].astype(jnp.int32)
    seg_pad = jnp.zeros((NPAD,), jnp.int32).at[:NNZ].set(seg)
    seg2d = seg_pad.reshape(NPAD // BLK, BLK)
    return _pool(seg2d, tens_values)
